# baseline (speedup 1.0000x reference)
import functools

import jax
import jax.numpy as jnp
from jax import lax
from jax.experimental import pallas as pl
from jax.experimental.pallas import tpu as pltpu
from jax.experimental.pallas import tpu_sc as plsc

_L = 16
_NW = 32
_CH = 128
_SENT_R = 1 << 18


def _tc_body(x_ref, val_ref, s_ref, mn_ref, mx_ref, acc_ref):
    i = pl.program_id(0)
    xb = x_ref[...]
    vb = val_ref[...]
    s_ref[...] = 1.0 / (1.0 + jnp.exp(-vb))
    bmn = jnp.min(xb)
    bmx = jnp.max(xb)

    @pl.when(i == 0)
    def _():
        acc_ref[0] = bmn
        acc_ref[1] = bmx

    @pl.when(i > 0)
    def _():
        acc_ref[0] = jnp.minimum(acc_ref[0], bmn)
        acc_ref[1] = jnp.maximum(acc_ref[1], bmx)

    @pl.when(i == pl.num_programs(0) - 1)
    def _():
        mn = acc_ref[0]
        mx = acc_ref[1]
        for j in range(_L):
            mn_ref[j] = mn
            mx_ref[j] = mx


def _tc_pass(x, val):
    m, d = x.shape
    b = val.shape[0]
    grid = 64
    xr = m // grid
    vr = b // grid
    return pl.pallas_call(
        _tc_body,
        grid=(grid,),
        in_specs=[
            pl.BlockSpec((xr, d), lambda i: (i, 0)),
            pl.BlockSpec((vr, d), lambda i: (i, 0)),
        ],
        out_specs=[
            pl.BlockSpec((vr, d), lambda i: (i, 0)),
            pl.BlockSpec(memory_space=pltpu.SMEM),
            pl.BlockSpec(memory_space=pltpu.SMEM),
        ],
        out_shape=[
            jax.ShapeDtypeStruct((b, d), jnp.float32),
            jax.ShapeDtypeStruct((_L,), jnp.float32),
            jax.ShapeDtypeStruct((_L,), jnp.float32),
        ],
        scratch_shapes=[pltpu.SMEM((2,), jnp.float32)],
        compiler_params=pltpu.CompilerParams(
            dimension_semantics=("arbitrary",),
        ),
    )(x, val)


def _sc_body(m, b, out_ref, idx_hbm, s_hbm, mn_hbm, mx_hbm,
             idx_v, selr_v, selb_v, finr_v, finb_v, pos_v,
             mn_v, mx_v, rchunk_v, bchunk_v, rowbuf_v, sem):
    rpw = m // _NW
    wid = lax.axis_index("s") * 2 + lax.axis_index("c")
    lo = wid * rpw
    iota = lax.iota(jnp.int32, _L)

    pltpu.sync_copy(idx_hbm, idx_v)
    pltpu.sync_copy(mn_hbm, mn_v)
    pltpu.sync_copy(mx_hbm, mx_v)

    def init_body(t, c):
        pos_v[pl.ds(t * _L, _L)] = jnp.full((_L,), -1, jnp.int32)
        return c

    lax.fori_loop(0, rpw // _L, init_body, 0)

    def sel_body(t, kc):
        v = idx_v[pl.ds(t * _L, _L)]
        sel = (v >= lo) & (v < lo + rpw)
        r = jnp.where(sel, v - lo, 0)
        bb = iota + t * _L
        csum = plsc.cumsum(jnp.where(sel, 1, 0))
        posn = kc + csum - 1
        plsc.store_scatter(selr_v, [posn], r, mask=sel)
        plsc.store_scatter(selb_v, [posn], bb, mask=sel)
        return kc + jnp.sum(jnp.where(sel, 1, 0))

    kc = lax.fori_loop(0, b // _L, sel_body, 0)

    selr_v[pl.ds(kc, _L)] = jnp.full((_L,), _SENT_R, jnp.int32)
    selb_v[pl.ds(kc, _L)] = iota

    nv2 = (kc + _L - 1) // _L

    def a2_body(j, c):
        r = selr_v[pl.ds(j * _L, _L)]
        bb = selb_v[pl.ds(j * _L, _L)]
        key = lax.sort(r * b + bb, dimension=0, is_stable=False)
        rs = key // b
        bs = key - rs * b
        nxt = rs.at[jnp.minimum(iota + 1, _L - 1)].get(
            mode="promise_in_bounds")
        islast = (iota == _L - 1) | (rs != nxt)
        mask = islast & (rs < rpw)
        rc = jnp.where(mask, rs, 0)
        plsc.store_scatter(pos_v, [rc], bs, mask=mask)
        return c

    lax.fori_loop(0, nv2, a2_body, 0)

    def b_body(j, c2):
        r = selr_v[pl.ds(j * _L, _L)]
        bb = selb_v[pl.ds(j * _L, _L)]
        valid = r < rpw
        rc = jnp.where(valid, r, 0)
        g = plsc.load_gather(pos_v, [rc])
        win = valid & (g == bb)
        csum = plsc.cumsum(jnp.where(win, 1, 0))
        posn = c2 + csum - 1
        plsc.store_scatter(finr_v, [posn], r + lo, mask=win)
        plsc.store_scatter(finb_v, [posn], bb, mask=win)
        return c2 + jnp.sum(jnp.where(win, 1, 0))

    c2 = lax.fori_loop(0, nv2, b_body, 0)

    @pl.when(c2 > 0)
    def _():
        zidx = jnp.zeros((_L,), jnp.int32)
        padr = finr_v[pl.ds(0, _L)].at[zidx].get(mode="promise_in_bounds")
        padb = finb_v[pl.ds(0, _L)].at[zidx].get(mode="promise_in_bounds")
        for u in range(_CH // _L):
            finr_v[pl.ds(c2 + u * _L, _L)] = padr
            finb_v[pl.ds(c2 + u * _L, _L)] = padb

        mnv = mn_v[...]
        mxv = mx_v[...]
        nch = (c2 + _CH - 1) // _CH

        def ch_body(t, c):
            for u in range(_CH // _L):
                rchunk_v[pl.ds(u * _L, _L)] = finr_v[pl.ds(t * _CH + u * _L, _L)]
                bchunk_v[pl.ds(u * _L, _L)] = finb_v[pl.ds(t * _CH + u * _L, _L)]
            pltpu.async_copy(s_hbm.at[bchunk_v], rowbuf_v, sem).wait()

            def row_body(rr, cc):
                for u in range(256 // _L):
                    sv = rowbuf_v[rr, pl.ds(u * _L, _L)]
                    rowbuf_v[rr, pl.ds(u * _L, _L)] = mnv + (mxv - mnv) * sv
                return cc

            lax.fori_loop(0, _CH, row_body, 0)
            pltpu.async_copy(rowbuf_v, out_ref.at[rchunk_v], sem).wait()
            return c

        lax.fori_loop(0, nch, ch_body, 0)


def _sc_scatter(out_mutref, idx, s, mn, mx):
    m = out_mutref.shape[0]
    b, d = s.shape
    mesh = plsc.VectorSubcoreMesh(core_axis_name="c", subcore_axis_name="s")
    body = functools.partial(_sc_body, m, b)
    pl.kernel(
        body,
        out_type=(),
        mesh=mesh,
        scratch_types=[
            pltpu.VMEM((b,), jnp.int32),
            pltpu.VMEM((b + _L,), jnp.int32),
            pltpu.VMEM((b + _L,), jnp.int32),
            pltpu.VMEM((b + _CH + _L,), jnp.int32),
            pltpu.VMEM((b + _CH + _L,), jnp.int32),
            pltpu.VMEM((m // _NW,), jnp.int32),
            pltpu.VMEM((_L,), jnp.float32),
            pltpu.VMEM((_L,), jnp.float32),
            pltpu.VMEM((_CH,), jnp.int32),
            pltpu.VMEM((_CH,), jnp.int32),
            pltpu.VMEM((_CH, d), jnp.float32),
            pltpu.SemaphoreType.DMA,
        ],
        compiler_params=pltpu.CompilerParams(needs_layout_passes=False),
    )(out_mutref, idx, s, mn, mx)


def kernel(x, idx, val):
    s, mn, mx = _tc_pass(x, val)
    ref = jax.new_ref(x)
    _sc_scatter(ref, idx.astype(jnp.int32), s, mn, mx)
    return jax.freeze(ref)


# speedup vs baseline: 1.2058x; 1.2058x over previous
import functools

import jax
import jax.numpy as jnp
from jax import lax
from jax.experimental import pallas as pl
from jax.experimental.pallas import tpu as pltpu
from jax.experimental.pallas import tpu_sc as plsc

_L = 16
_NW = 32
_CH = 128
_SENT_R = 1 << 18


def _tc_body(x_ref, val_ref, out_ref, s_ref, mn_ref, mx_ref, acc_ref):
    i = pl.program_id(0)
    xb = x_ref[...]
    out_ref[...] = xb
    vb = val_ref[...]
    s_ref[...] = 1.0 / (1.0 + jnp.exp(-vb))
    bmn = jnp.min(xb)
    bmx = jnp.max(xb)

    @pl.when(i == 0)
    def _():
        acc_ref[0] = bmn
        acc_ref[1] = bmx

    @pl.when(i > 0)
    def _():
        acc_ref[0] = jnp.minimum(acc_ref[0], bmn)
        acc_ref[1] = jnp.maximum(acc_ref[1], bmx)

    @pl.when(i == pl.num_programs(0) - 1)
    def _():
        mn = acc_ref[0]
        mx = acc_ref[1]
        for j in range(_L):
            mn_ref[j] = mn
            mx_ref[j] = mx


def _tc_pass(x, val):
    m, d = x.shape
    b = val.shape[0]
    grid = 64
    xr = m // grid
    vr = b // grid
    return pl.pallas_call(
        _tc_body,
        grid=(grid,),
        in_specs=[
            pl.BlockSpec((xr, d), lambda i: (i, 0)),
            pl.BlockSpec((vr, d), lambda i: (i, 0)),
        ],
        out_specs=[
            pl.BlockSpec((xr, d), lambda i: (i, 0)),
            pl.BlockSpec((vr, d), lambda i: (i, 0)),
            pl.BlockSpec(memory_space=pltpu.SMEM),
            pl.BlockSpec(memory_space=pltpu.SMEM),
        ],
        out_shape=[
            jax.ShapeDtypeStruct((m, d), jnp.float32),
            jax.ShapeDtypeStruct((b, d), jnp.float32),
            jax.ShapeDtypeStruct((_L,), jnp.float32),
            jax.ShapeDtypeStruct((_L,), jnp.float32),
        ],
        scratch_shapes=[pltpu.SMEM((2,), jnp.float32)],
        compiler_params=pltpu.CompilerParams(
            dimension_semantics=("arbitrary",),
        ),
    )(x, val)


_FW = 4352


def _s1_body(m, b, idx_hbm, finr_hbm, finb_hbm, cnt_hbm,
             idx_v, selr_v, selb_v, finr_v, finb_v, pos_v, cnt_v):
    rpw = m // _NW
    wid = lax.axis_index("s") * 2 + lax.axis_index("c")
    lo = wid * rpw
    iota = lax.iota(jnp.int32, _L)

    pltpu.sync_copy(idx_hbm, idx_v)

    def init_body(t, c):
        pos_v[pl.ds(t * _L, _L)] = jnp.full((_L,), -1, jnp.int32)
        return c

    lax.fori_loop(0, rpw // _L, init_body, 0)

    def sel_body(t, kc):
        v = idx_v[pl.ds(t * _L, _L)]
        sel = (v >= lo) & (v < lo + rpw)
        r = jnp.where(sel, v - lo, 0)
        bb = iota + t * _L
        csum = plsc.cumsum(jnp.where(sel, 1, 0))
        posn = kc + csum - 1
        plsc.store_scatter(selr_v, [posn], r, mask=sel)
        plsc.store_scatter(selb_v, [posn], bb, mask=sel)
        return kc + jnp.sum(jnp.where(sel, 1, 0))

    kc = lax.fori_loop(0, b // _L, sel_body, 0)

    selr_v[pl.ds(kc, _L)] = jnp.full((_L,), _SENT_R, jnp.int32)
    selb_v[pl.ds(kc, _L)] = iota

    nv2 = (kc + _L - 1) // _L

    def a2_body(j, c):
        r = selr_v[pl.ds(j * _L, _L)]
        bb = selb_v[pl.ds(j * _L, _L)]
        key = lax.sort(r * b + bb, dimension=0, is_stable=False)
        rs = key // b
        bs = key - rs * b
        nxt = rs.at[jnp.minimum(iota + 1, _L - 1)].get(
            mode="promise_in_bounds")
        islast = (iota == _L - 1) | (rs != nxt)
        mask = islast & (rs < rpw)
        rc = jnp.where(mask, rs, 0)
        plsc.store_scatter(pos_v, [rc], bs, mask=mask)
        return c

    lax.fori_loop(0, nv2, a2_body, 0)

    def b_body(j, c2):
        r = selr_v[pl.ds(j * _L, _L)]
        bb = selb_v[pl.ds(j * _L, _L)]
        valid = r < rpw
        rc = jnp.where(valid, r, 0)
        g = plsc.load_gather(pos_v, [rc])
        win = valid & (g == bb)
        csum = plsc.cumsum(jnp.where(win, 1, 0))
        posn = c2 + csum - 1
        plsc.store_scatter(finr_v, [posn], r + lo, mask=win)
        plsc.store_scatter(finb_v, [posn], bb, mask=win)
        return c2 + jnp.sum(jnp.where(win, 1, 0))

    c2 = lax.fori_loop(0, nv2, b_body, 0)

    @pl.when(c2 > 0)
    def _():
        zidx = jnp.zeros((_L,), jnp.int32)
        padr = finr_v[pl.ds(0, _L)].at[zidx].get(mode="promise_in_bounds")
        padb = finb_v[pl.ds(0, _L)].at[zidx].get(mode="promise_in_bounds")
        for u in range(_CH // _L):
            finr_v[pl.ds(c2 + u * _L, _L)] = padr
            finb_v[pl.ds(c2 + u * _L, _L)] = padb

    cnt_v[pl.ds(0, _L)] = jnp.full((_L,), c2, jnp.int32)
    pltpu.sync_copy(finr_v, finr_hbm.at[wid])
    pltpu.sync_copy(finb_v, finb_hbm.at[wid])
    pltpu.sync_copy(cnt_v, cnt_hbm.at[wid])


def _s1_select(idx, m):
    b = idx.shape[0]
    mesh = plsc.VectorSubcoreMesh(core_axis_name="c", subcore_axis_name="s")
    body = functools.partial(_s1_body, m, b)
    return pl.kernel(
        body,
        out_type=[
            jax.ShapeDtypeStruct((_NW, _FW), jnp.int32),
            jax.ShapeDtypeStruct((_NW, _FW), jnp.int32),
            jax.ShapeDtypeStruct((_NW, _L), jnp.int32),
        ],
        mesh=mesh,
        scratch_types=[
            pltpu.VMEM((b,), jnp.int32),
            pltpu.VMEM((_FW,), jnp.int32),
            pltpu.VMEM((_FW,), jnp.int32),
            pltpu.VMEM((_FW,), jnp.int32),
            pltpu.VMEM((_FW,), jnp.int32),
            pltpu.VMEM((m // _NW,), jnp.int32),
            pltpu.VMEM((_L,), jnp.int32),
        ],
        compiler_params=pltpu.CompilerParams(needs_layout_passes=False),
    )(idx)


def _s2_body(d, out_ref, finr_hbm, finb_hbm, cnt_hbm, s_hbm, mn_hbm, mx_hbm,
             finr_v, finb_v, cnt_v, mn_v, mx_v, rchunk_v, bchunk_v,
             rowbuf_v, sem):
    wid = lax.axis_index("s") * 2 + lax.axis_index("c")
    pltpu.sync_copy(finr_hbm.at[wid], finr_v)
    pltpu.sync_copy(finb_hbm.at[wid], finb_v)
    pltpu.sync_copy(cnt_hbm.at[wid], cnt_v)
    pltpu.sync_copy(mn_hbm, mn_v)
    pltpu.sync_copy(mx_hbm, mx_v)
    c2 = jnp.max(cnt_v[pl.ds(0, _L)])

    @pl.when(c2 > 0)
    def _():
        mnv = mn_v[...]
        mxv = mx_v[...]
        nch = (c2 + _CH - 1) // _CH

        def ch_body(t, c):
            for u in range(_CH // _L):
                rchunk_v[pl.ds(u * _L, _L)] = finr_v[pl.ds(t * _CH + u * _L, _L)]
                bchunk_v[pl.ds(u * _L, _L)] = finb_v[pl.ds(t * _CH + u * _L, _L)]
            pltpu.async_copy(s_hbm.at[bchunk_v], rowbuf_v, sem).wait()

            def row_body(rr, cc):
                for u in range(d // _L):
                    sv = rowbuf_v[rr, pl.ds(u * _L, _L)]
                    rowbuf_v[rr, pl.ds(u * _L, _L)] = mnv + (mxv - mnv) * sv
                return cc

            lax.fori_loop(0, _CH, row_body, 0)
            pltpu.async_copy(rowbuf_v, out_ref.at[rchunk_v], sem).wait()
            return c

        lax.fori_loop(0, nch, ch_body, 0)


def _s2_scatter(out_mutref, finr, finb, cnt, s, mn, mx):
    d = s.shape[1]
    mesh = plsc.VectorSubcoreMesh(core_axis_name="c", subcore_axis_name="s")
    body = functools.partial(_s2_body, d)
    pl.kernel(
        body,
        out_type=(),
        mesh=mesh,
        scratch_types=[
            pltpu.VMEM((_FW,), jnp.int32),
            pltpu.VMEM((_FW,), jnp.int32),
            pltpu.VMEM((_L,), jnp.int32),
            pltpu.VMEM((_L,), jnp.float32),
            pltpu.VMEM((_L,), jnp.float32),
            pltpu.VMEM((_CH,), jnp.int32),
            pltpu.VMEM((_CH,), jnp.int32),
            pltpu.VMEM((_CH, d), jnp.float32),
            pltpu.SemaphoreType.DMA,
        ],
        compiler_params=pltpu.CompilerParams(needs_layout_passes=False),
    )(out_mutref, finr, finb, cnt, s, mn, mx)


def kernel(x, idx, val):
    finr, finb, cnt = _s1_select(idx.astype(jnp.int32), x.shape[0])
    out0, s, mn, mx = _tc_pass(x, val)
    ref = jax.new_ref(out0)
    _s2_scatter(ref, finr, finb, cnt, s, mn, mx)
    return jax.freeze(ref)


# speedup vs baseline: 1.4556x; 1.2072x over previous
import functools

import jax
import jax.numpy as jnp
from jax import lax
from jax.experimental import pallas as pl
from jax.experimental.pallas import tpu as pltpu
from jax.experimental.pallas import tpu_sc as plsc

_L = 16
_NW = 32
_CH = 128
_SENT_R = 1 << 18


def _tc_body(x_ref, val_ref, out_ref, s_ref, mn_ref, mx_ref, acc_ref):
    i = pl.program_id(0)
    xb = x_ref[...]
    out_ref[...] = xb
    vb = val_ref[...]
    s_ref[...] = 1.0 / (1.0 + jnp.exp(-vb))
    bmn = jnp.min(xb)
    bmx = jnp.max(xb)

    @pl.when(i == 0)
    def _():
        acc_ref[0] = bmn
        acc_ref[1] = bmx

    @pl.when(i > 0)
    def _():
        acc_ref[0] = jnp.minimum(acc_ref[0], bmn)
        acc_ref[1] = jnp.maximum(acc_ref[1], bmx)

    @pl.when(i == pl.num_programs(0) - 1)
    def _():
        mn = acc_ref[0]
        mx = acc_ref[1]
        for j in range(_L):
            mn_ref[j] = mn
            mx_ref[j] = mx


def _tc_pass(x, val):
    m, d = x.shape
    b = val.shape[0]
    grid = 32
    xr = m // grid
    vr = b // grid
    return pl.pallas_call(
        _tc_body,
        grid=(grid,),
        in_specs=[
            pl.BlockSpec((xr, d), lambda i: (i, 0)),
            pl.BlockSpec((vr, d), lambda i: (i, 0)),
        ],
        out_specs=[
            pl.BlockSpec((xr, d), lambda i: (i, 0)),
            pl.BlockSpec((vr, d), lambda i: (i, 0)),
            pl.BlockSpec(memory_space=pltpu.SMEM),
            pl.BlockSpec(memory_space=pltpu.SMEM),
        ],
        out_shape=[
            jax.ShapeDtypeStruct((m, d), jnp.float32),
            jax.ShapeDtypeStruct((b, d), jnp.float32),
            jax.ShapeDtypeStruct((_L,), jnp.float32),
            jax.ShapeDtypeStruct((_L,), jnp.float32),
        ],
        scratch_shapes=[pltpu.SMEM((2,), jnp.float32)],
        compiler_params=pltpu.CompilerParams(
            dimension_semantics=("arbitrary",),
        ),
    )(x, val)


_FW = 4352


def _s1_body(m, b, idx_hbm, finr_hbm, finb_hbm, cnt_hbm,
             idx_v, selr_v, selb_v, finr_v, finb_v, pos_v, cnt_v):
    rpw = m // _NW
    wid = lax.axis_index("s") * 2 + lax.axis_index("c")
    lo = wid * rpw
    iota = lax.iota(jnp.int32, _L)

    pltpu.sync_copy(idx_hbm, idx_v)

    def sel_body(t, kc):
        v = idx_v[pl.ds(t * _L, _L)]
        sel = (v >= lo) & (v < lo + rpw)
        r = jnp.where(sel, v - lo, 0)
        bb = iota + t * _L
        csum = plsc.cumsum(jnp.where(sel, 1, 0))
        posn = kc + csum - 1
        plsc.store_scatter(selr_v, [posn], r, mask=sel)
        plsc.store_scatter(selb_v, [posn], bb, mask=sel)
        return kc + jnp.sum(jnp.where(sel, 1, 0))

    kc = lax.fori_loop(0, b // _L, sel_body, 0)

    selr_v[pl.ds(kc, _L)] = jnp.full((_L,), _SENT_R, jnp.int32)
    selb_v[pl.ds(kc, _L)] = iota

    nv2 = (kc + _L - 1) // _L

    def a2_body(j, c):
        r = selr_v[pl.ds(j * _L, _L)]
        bb = selb_v[pl.ds(j * _L, _L)]
        key = lax.sort(r * b + bb, dimension=0, is_stable=False)
        rs = key // b
        bs = key - rs * b
        nxt = rs.at[jnp.minimum(iota + 1, _L - 1)].get(
            mode="promise_in_bounds")
        islast = (iota == _L - 1) | (rs != nxt)
        mask = islast & (rs < rpw)
        rc = jnp.where(mask, rs, 0)
        plsc.store_scatter(pos_v, [rc], bs, mask=mask)
        return c

    lax.fori_loop(0, nv2, a2_body, 0)

    def b_body(j, c2):
        r = selr_v[pl.ds(j * _L, _L)]
        bb = selb_v[pl.ds(j * _L, _L)]
        valid = r < rpw
        rc = jnp.where(valid, r, 0)
        g = plsc.load_gather(pos_v, [rc])
        win = valid & (g == bb)
        csum = plsc.cumsum(jnp.where(win, 1, 0))
        posn = c2 + csum - 1
        plsc.store_scatter(finr_v, [posn], r + lo, mask=win)
        plsc.store_scatter(finb_v, [posn], bb, mask=win)
        return c2 + jnp.sum(jnp.where(win, 1, 0))

    c2 = lax.fori_loop(0, nv2, b_body, 0)

    @pl.when(c2 > 0)
    def _():
        zidx = jnp.zeros((_L,), jnp.int32)
        padr = finr_v[pl.ds(0, _L)].at[zidx].get(mode="promise_in_bounds")
        padb = finb_v[pl.ds(0, _L)].at[zidx].get(mode="promise_in_bounds")
        for u in range(_CH // _L):
            finr_v[pl.ds(c2 + u * _L, _L)] = padr
            finb_v[pl.ds(c2 + u * _L, _L)] = padb

    cnt_v[pl.ds(0, _L)] = jnp.full((_L,), c2, jnp.int32)
    pltpu.sync_copy(finr_v, finr_hbm.at[wid])
    pltpu.sync_copy(finb_v, finb_hbm.at[wid])
    pltpu.sync_copy(cnt_v, cnt_hbm.at[wid])


def _s1_select(idx, m):
    b = idx.shape[0]
    mesh = plsc.VectorSubcoreMesh(core_axis_name="c", subcore_axis_name="s")
    body = functools.partial(_s1_body, m, b)
    return pl.kernel(
        body,
        out_type=[
            jax.ShapeDtypeStruct((_NW, _FW), jnp.int32),
            jax.ShapeDtypeStruct((_NW, _FW), jnp.int32),
            jax.ShapeDtypeStruct((_NW, _L), jnp.int32),
        ],
        mesh=mesh,
        scratch_types=[
            pltpu.VMEM((b,), jnp.int32),
            pltpu.VMEM((_FW,), jnp.int32),
            pltpu.VMEM((_FW,), jnp.int32),
            pltpu.VMEM((_FW,), jnp.int32),
            pltpu.VMEM((_FW,), jnp.int32),
            pltpu.VMEM((m // _NW,), jnp.int32),
            pltpu.VMEM((_L,), jnp.int32),
        ],
        compiler_params=pltpu.CompilerParams(needs_layout_passes=False),
    )(idx)


def _s2_body(d, out_ref, finr_hbm, finb_hbm, cnt_hbm, s_hbm, mn_hbm, mx_hbm,
             finr_v, finb_v, cnt_v, mn_v, mx_v, rchunk_v, bchunk_v,
             rowbuf_v, sem):
    wid = lax.axis_index("s") * 2 + lax.axis_index("c")
    pltpu.sync_copy(finr_hbm.at[wid], finr_v)
    pltpu.sync_copy(finb_hbm.at[wid], finb_v)
    pltpu.sync_copy(cnt_hbm.at[wid], cnt_v)
    pltpu.sync_copy(mn_hbm, mn_v)
    pltpu.sync_copy(mx_hbm, mx_v)
    c2 = jnp.max(cnt_v[pl.ds(0, _L)])

    @pl.when(c2 > 0)
    def _():
        mnv = mn_v[...]
        mxv = mx_v[...]
        nch = (c2 + _CH - 1) // _CH

        def ch_body(t, c):
            for u in range(_CH // _L):
                rchunk_v[pl.ds(u * _L, _L)] = finr_v[pl.ds(t * _CH + u * _L, _L)]
                bchunk_v[pl.ds(u * _L, _L)] = finb_v[pl.ds(t * _CH + u * _L, _L)]
            pltpu.async_copy(s_hbm.at[bchunk_v], rowbuf_v, sem).wait()

            def row_body(rr, cc):
                for u in range(d // _L):
                    sv = rowbuf_v[rr, pl.ds(u * _L, _L)]
                    rowbuf_v[rr, pl.ds(u * _L, _L)] = mnv + (mxv - mnv) * sv
                return cc

            lax.fori_loop(0, _CH, row_body, 0)
            pltpu.async_copy(rowbuf_v, out_ref.at[rchunk_v], sem).wait()
            return c

        lax.fori_loop(0, nch, ch_body, 0)


def _s2_scatter(out_mutref, finr, finb, cnt, s, mn, mx):
    d = s.shape[1]
    mesh = plsc.VectorSubcoreMesh(core_axis_name="c", subcore_axis_name="s")
    body = functools.partial(_s2_body, d)
    pl.kernel(
        body,
        out_type=(),
        mesh=mesh,
        scratch_types=[
            pltpu.VMEM((_FW,), jnp.int32),
            pltpu.VMEM((_FW,), jnp.int32),
            pltpu.VMEM((_L,), jnp.int32),
            pltpu.VMEM((_L,), jnp.float32),
            pltpu.VMEM((_L,), jnp.float32),
            pltpu.VMEM((_CH,), jnp.int32),
            pltpu.VMEM((_CH,), jnp.int32),
            pltpu.VMEM((_CH, d), jnp.float32),
            pltpu.SemaphoreType.DMA,
        ],
        compiler_params=pltpu.CompilerParams(needs_layout_passes=False),
    )(out_mutref, finr, finb, cnt, s, mn, mx)


def kernel(x, idx, val):
    finr, finb, cnt = _s1_select(idx.astype(jnp.int32), x.shape[0])
    out0, s, mn, mx = _tc_pass(x, val)
    ref = jax.new_ref(out0)
    _s2_scatter(ref, finr, finb, cnt, s, mn, mx)
    return jax.freeze(ref)


# speedup vs baseline: 1.5391x; 1.0573x over previous
import functools

import jax
import jax.numpy as jnp
from jax import lax
from jax.experimental import pallas as pl
from jax.experimental.pallas import tpu as pltpu
from jax.experimental.pallas import tpu_sc as plsc

_L = 16
_NW = 32
_CH = 128
_SENT_R = 1 << 18


def _tc_body(x_ref, val_ref, out_ref, s_ref, mn_ref, mx_ref, acc_ref):
    i = pl.program_id(0)
    xb = x_ref[...]
    out_ref[...] = xb
    vb = val_ref[...]
    s_ref[...] = 1.0 / (1.0 + jnp.exp(-vb))
    bmn = jnp.min(xb)
    bmx = jnp.max(xb)

    @pl.when(i == 0)
    def _():
        acc_ref[0] = bmn
        acc_ref[1] = bmx

    @pl.when(i > 0)
    def _():
        acc_ref[0] = jnp.minimum(acc_ref[0], bmn)
        acc_ref[1] = jnp.maximum(acc_ref[1], bmx)

    @pl.when(i == pl.num_programs(0) - 1)
    def _():
        mn = acc_ref[0]
        mx = acc_ref[1]
        for j in range(_L):
            mn_ref[j] = mn
            mx_ref[j] = mx


def _tc_pass(x, val):
    m, d = x.shape
    b = val.shape[0]
    grid = 16
    xr = m // grid
    vr = b // grid
    return pl.pallas_call(
        _tc_body,
        grid=(grid,),
        in_specs=[
            pl.BlockSpec((xr, d), lambda i: (i, 0)),
            pl.BlockSpec((vr, d), lambda i: (i, 0)),
        ],
        out_specs=[
            pl.BlockSpec((xr, d), lambda i: (i, 0)),
            pl.BlockSpec((vr, d), lambda i: (i, 0)),
            pl.BlockSpec(memory_space=pltpu.SMEM),
            pl.BlockSpec(memory_space=pltpu.SMEM),
        ],
        out_shape=[
            jax.ShapeDtypeStruct((m, d), jnp.float32),
            jax.ShapeDtypeStruct((b, d), jnp.float32),
            jax.ShapeDtypeStruct((_L,), jnp.float32),
            jax.ShapeDtypeStruct((_L,), jnp.float32),
        ],
        scratch_shapes=[pltpu.SMEM((2,), jnp.float32)],
        compiler_params=pltpu.CompilerParams(
            dimension_semantics=("arbitrary",),
        ),
    )(x, val)


_FW = 4352


def _s1_body(m, b, idx_hbm, finr_hbm, finb_hbm, cnt_hbm,
             idx_v, selr_v, selb_v, finr_v, finb_v, pos_v, cnt_v):
    rpw = m // _NW
    wid = lax.axis_index("s") * 2 + lax.axis_index("c")
    lo = wid * rpw
    iota = lax.iota(jnp.int32, _L)

    pltpu.sync_copy(idx_hbm, idx_v)

    def sel_body(t, kc):
        v = idx_v[pl.ds(t * _L, _L)]
        sel = (v >= lo) & (v < lo + rpw)
        r = jnp.where(sel, v - lo, 0)
        bb = iota + t * _L
        csum = plsc.cumsum(jnp.where(sel, 1, 0))
        posn = kc + csum - 1
        plsc.store_scatter(selr_v, [posn], r, mask=sel)
        plsc.store_scatter(selb_v, [posn], bb, mask=sel)
        return kc + jnp.sum(jnp.where(sel, 1, 0))

    kc = lax.fori_loop(0, b // _L, sel_body, 0)

    selr_v[pl.ds(kc, _L)] = jnp.full((_L,), _SENT_R, jnp.int32)
    selb_v[pl.ds(kc, _L)] = iota

    nv2 = (kc + _L - 1) // _L

    def a2_body(j, c):
        r = selr_v[pl.ds(j * _L, _L)]
        bb = selb_v[pl.ds(j * _L, _L)]
        key = lax.sort(r * b + bb, dimension=0, is_stable=False)
        rs = key // b
        bs = key - rs * b
        nxt = rs.at[jnp.minimum(iota + 1, _L - 1)].get(
            mode="promise_in_bounds")
        islast = (iota == _L - 1) | (rs != nxt)
        mask = islast & (rs < rpw)
        rc = jnp.where(mask, rs, 0)
        plsc.store_scatter(pos_v, [rc], bs, mask=mask)
        return c

    lax.fori_loop(0, nv2, a2_body, 0)

    def b_body(j, c2):
        r = selr_v[pl.ds(j * _L, _L)]
        bb = selb_v[pl.ds(j * _L, _L)]
        valid = r < rpw
        rc = jnp.where(valid, r, 0)
        g = plsc.load_gather(pos_v, [rc])
        win = valid & (g == bb)
        csum = plsc.cumsum(jnp.where(win, 1, 0))
        posn = c2 + csum - 1
        plsc.store_scatter(finr_v, [posn], r + lo, mask=win)
        plsc.store_scatter(finb_v, [posn], bb, mask=win)
        return c2 + jnp.sum(jnp.where(win, 1, 0))

    c2 = lax.fori_loop(0, nv2, b_body, 0)

    @pl.when(c2 > 0)
    def _():
        zidx = jnp.zeros((_L,), jnp.int32)
        padr = finr_v[pl.ds(0, _L)].at[zidx].get(mode="promise_in_bounds")
        padb = finb_v[pl.ds(0, _L)].at[zidx].get(mode="promise_in_bounds")
        for u in range(_CH // _L):
            finr_v[pl.ds(c2 + u * _L, _L)] = padr
            finb_v[pl.ds(c2 + u * _L, _L)] = padb

    cnt_v[pl.ds(0, _L)] = jnp.full((_L,), c2, jnp.int32)
    pltpu.sync_copy(finr_v, finr_hbm.at[wid])
    pltpu.sync_copy(finb_v, finb_hbm.at[wid])
    pltpu.sync_copy(cnt_v, cnt_hbm.at[wid])


def _s1_select(idx, m):
    b = idx.shape[0]
    mesh = plsc.VectorSubcoreMesh(core_axis_name="c", subcore_axis_name="s")
    body = functools.partial(_s1_body, m, b)
    return pl.kernel(
        body,
        out_type=[
            jax.ShapeDtypeStruct((_NW, _FW), jnp.int32),
            jax.ShapeDtypeStruct((_NW, _FW), jnp.int32),
            jax.ShapeDtypeStruct((_NW, _L), jnp.int32),
        ],
        mesh=mesh,
        scratch_types=[
            pltpu.VMEM((b,), jnp.int32),
            pltpu.VMEM((_FW,), jnp.int32),
            pltpu.VMEM((_FW,), jnp.int32),
            pltpu.VMEM((_FW,), jnp.int32),
            pltpu.VMEM((_FW,), jnp.int32),
            pltpu.VMEM((m // _NW,), jnp.int32),
            pltpu.VMEM((_L,), jnp.int32),
        ],
        compiler_params=pltpu.CompilerParams(needs_layout_passes=False),
    )(idx)


def _s2_body(d, out_ref, finr_hbm, finb_hbm, cnt_hbm, s_hbm, mn_hbm, mx_hbm,
             finr_v, finb_v, cnt_v, mn_v, mx_v, rchunk_v, bchunk_v,
             rowbuf_v, sem):
    wid = lax.axis_index("s") * 2 + lax.axis_index("c")
    pltpu.sync_copy(finr_hbm.at[wid], finr_v)
    pltpu.sync_copy(finb_hbm.at[wid], finb_v)
    pltpu.sync_copy(cnt_hbm.at[wid], cnt_v)
    pltpu.sync_copy(mn_hbm, mn_v)
    pltpu.sync_copy(mx_hbm, mx_v)
    c2 = jnp.max(cnt_v[pl.ds(0, _L)])

    @pl.when(c2 > 0)
    def _():
        mnv = mn_v[...]
        mxv = mx_v[...]
        nch = (c2 + _CH - 1) // _CH

        def ch_body(t, c):
            for u in range(_CH // _L):
                rchunk_v[pl.ds(u * _L, _L)] = finr_v[pl.ds(t * _CH + u * _L, _L)]
                bchunk_v[pl.ds(u * _L, _L)] = finb_v[pl.ds(t * _CH + u * _L, _L)]
            pltpu.async_copy(s_hbm.at[bchunk_v], rowbuf_v, sem).wait()

            def row_body(rr, cc):
                for u in range(d // _L):
                    sv = rowbuf_v[rr, pl.ds(u * _L, _L)]
                    rowbuf_v[rr, pl.ds(u * _L, _L)] = mnv + (mxv - mnv) * sv
                return cc

            lax.fori_loop(0, _CH, row_body, 0)
            pltpu.async_copy(rowbuf_v, out_ref.at[rchunk_v], sem).wait()
            return c

        lax.fori_loop(0, nch, ch_body, 0)


def _s2_scatter(out_mutref, finr, finb, cnt, s, mn, mx):
    d = s.shape[1]
    mesh = plsc.VectorSubcoreMesh(core_axis_name="c", subcore_axis_name="s")
    body = functools.partial(_s2_body, d)
    pl.kernel(
        body,
        out_type=(),
        mesh=mesh,
        scratch_types=[
            pltpu.VMEM((_FW,), jnp.int32),
            pltpu.VMEM((_FW,), jnp.int32),
            pltpu.VMEM((_L,), jnp.int32),
            pltpu.VMEM((_L,), jnp.float32),
            pltpu.VMEM((_L,), jnp.float32),
            pltpu.VMEM((_CH,), jnp.int32),
            pltpu.VMEM((_CH,), jnp.int32),
            pltpu.VMEM((_CH, d), jnp.float32),
            pltpu.SemaphoreType.DMA,
        ],
        compiler_params=pltpu.CompilerParams(needs_layout_passes=False),
    )(out_mutref, finr, finb, cnt, s, mn, mx)


def kernel(x, idx, val):
    finr, finb, cnt = _s1_select(idx.astype(jnp.int32), x.shape[0])
    out0, s, mn, mx = _tc_pass(x, val)
    ref = jax.new_ref(out0)
    _s2_scatter(ref, finr, finb, cnt, s, mn, mx)
    return jax.freeze(ref)


# speedup vs baseline: 1.5986x; 1.0387x over previous
import functools

import jax
import jax.numpy as jnp
from jax import lax
from jax.experimental import pallas as pl
from jax.experimental.pallas import tpu as pltpu
from jax.experimental.pallas import tpu_sc as plsc

_L = 16
_NW = 32
_CH = 128
_SENT_R = 1 << 18


def _tc_body(x_ref, val_ref, out_ref, s_ref, mn_ref, mx_ref, acc_ref):
    i = pl.program_id(0)
    xb = x_ref[...]
    out_ref[...] = xb
    vb = val_ref[...]
    s_ref[...] = 1.0 / (1.0 + jnp.exp(-vb))
    bmn = jnp.min(xb)
    bmx = jnp.max(xb)

    @pl.when(i == 0)
    def _():
        acc_ref[0] = bmn
        acc_ref[1] = bmx

    @pl.when(i > 0)
    def _():
        acc_ref[0] = jnp.minimum(acc_ref[0], bmn)
        acc_ref[1] = jnp.maximum(acc_ref[1], bmx)

    @pl.when(i == pl.num_programs(0) - 1)
    def _():
        mn = acc_ref[0]
        mx = acc_ref[1]
        for j in range(_L):
            mn_ref[j] = mn
            mx_ref[j] = mx


def _tc_pass(x, val):
    m, d = x.shape
    b = val.shape[0]
    grid = 8
    xr = m // grid
    vr = b // grid
    return pl.pallas_call(
        _tc_body,
        grid=(grid,),
        in_specs=[
            pl.BlockSpec((xr, d), lambda i: (i, 0)),
            pl.BlockSpec((vr, d), lambda i: (i, 0)),
        ],
        out_specs=[
            pl.BlockSpec((xr, d), lambda i: (i, 0)),
            pl.BlockSpec((vr, d), lambda i: (i, 0)),
            pl.BlockSpec(memory_space=pltpu.SMEM),
            pl.BlockSpec(memory_space=pltpu.SMEM),
        ],
        out_shape=[
            jax.ShapeDtypeStruct((m, d), jnp.float32),
            jax.ShapeDtypeStruct((b, d), jnp.float32),
            jax.ShapeDtypeStruct((_L,), jnp.float32),
            jax.ShapeDtypeStruct((_L,), jnp.float32),
        ],
        scratch_shapes=[pltpu.SMEM((2,), jnp.float32)],
        compiler_params=pltpu.CompilerParams(
            dimension_semantics=("arbitrary",),
        ),
    )(x, val)


_FW = 4352


def _s1_body(m, b, idx_hbm, finr_hbm, finb_hbm, cnt_hbm,
             idx_v, selr_v, selb_v, finr_v, finb_v, pos_v, cnt_v):
    rpw = m // _NW
    wid = lax.axis_index("s") * 2 + lax.axis_index("c")
    lo = wid * rpw
    iota = lax.iota(jnp.int32, _L)

    pltpu.sync_copy(idx_hbm, idx_v)

    def sel_body(t, kc):
        v = idx_v[pl.ds(t * _L, _L)]
        sel = (v >= lo) & (v < lo + rpw)
        r = jnp.where(sel, v - lo, 0)
        bb = iota + t * _L
        csum = plsc.cumsum(jnp.where(sel, 1, 0))
        posn = kc + csum - 1
        plsc.store_scatter(selr_v, [posn], r, mask=sel)
        plsc.store_scatter(selb_v, [posn], bb, mask=sel)
        return kc + jnp.sum(jnp.where(sel, 1, 0))

    kc = lax.fori_loop(0, b // _L, sel_body, 0)

    selr_v[pl.ds(kc, _L)] = jnp.full((_L,), _SENT_R, jnp.int32)
    selb_v[pl.ds(kc, _L)] = iota

    nv2 = (kc + _L - 1) // _L

    def a2_body(j, c):
        r = selr_v[pl.ds(j * _L, _L)]
        bb = selb_v[pl.ds(j * _L, _L)]
        key = lax.sort(r * b + bb, dimension=0, is_stable=False)
        rs = key // b
        bs = key - rs * b
        nxt = rs.at[jnp.minimum(iota + 1, _L - 1)].get(
            mode="promise_in_bounds")
        islast = (iota == _L - 1) | (rs != nxt)
        mask = islast & (rs < rpw)
        rc = jnp.where(mask, rs, 0)
        plsc.store_scatter(pos_v, [rc], bs, mask=mask)
        return c

    lax.fori_loop(0, nv2, a2_body, 0)

    def b_body(j, c2):
        r = selr_v[pl.ds(j * _L, _L)]
        bb = selb_v[pl.ds(j * _L, _L)]
        valid = r < rpw
        rc = jnp.where(valid, r, 0)
        g = plsc.load_gather(pos_v, [rc])
        win = valid & (g == bb)
        csum = plsc.cumsum(jnp.where(win, 1, 0))
        posn = c2 + csum - 1
        plsc.store_scatter(finr_v, [posn], r + lo, mask=win)
        plsc.store_scatter(finb_v, [posn], bb, mask=win)
        return c2 + jnp.sum(jnp.where(win, 1, 0))

    c2 = lax.fori_loop(0, nv2, b_body, 0)

    @pl.when(c2 > 0)
    def _():
        zidx = jnp.zeros((_L,), jnp.int32)
        padr = finr_v[pl.ds(0, _L)].at[zidx].get(mode="promise_in_bounds")
        padb = finb_v[pl.ds(0, _L)].at[zidx].get(mode="promise_in_bounds")
        for u in range(_CH // _L):
            finr_v[pl.ds(c2 + u * _L, _L)] = padr
            finb_v[pl.ds(c2 + u * _L, _L)] = padb

    cnt_v[pl.ds(0, _L)] = jnp.full((_L,), c2, jnp.int32)
    pltpu.sync_copy(finr_v, finr_hbm.at[wid])
    pltpu.sync_copy(finb_v, finb_hbm.at[wid])
    pltpu.sync_copy(cnt_v, cnt_hbm.at[wid])


def _s1_select(idx, m):
    b = idx.shape[0]
    mesh = plsc.VectorSubcoreMesh(core_axis_name="c", subcore_axis_name="s")
    body = functools.partial(_s1_body, m, b)
    return pl.kernel(
        body,
        out_type=[
            jax.ShapeDtypeStruct((_NW, _FW), jnp.int32),
            jax.ShapeDtypeStruct((_NW, _FW), jnp.int32),
            jax.ShapeDtypeStruct((_NW, _L), jnp.int32),
        ],
        mesh=mesh,
        scratch_types=[
            pltpu.VMEM((b,), jnp.int32),
            pltpu.VMEM((_FW,), jnp.int32),
            pltpu.VMEM((_FW,), jnp.int32),
            pltpu.VMEM((_FW,), jnp.int32),
            pltpu.VMEM((_FW,), jnp.int32),
            pltpu.VMEM((m // _NW,), jnp.int32),
            pltpu.VMEM((_L,), jnp.int32),
        ],
        compiler_params=pltpu.CompilerParams(needs_layout_passes=False),
    )(idx)


def _s2_body(d, out_ref, finr_hbm, finb_hbm, cnt_hbm, s_hbm, mn_hbm, mx_hbm,
             finr_v, finb_v, cnt_v, mn_v, mx_v, rchunk_v, bchunk_v,
             rowbuf_v, sem):
    wid = lax.axis_index("s") * 2 + lax.axis_index("c")
    pltpu.sync_copy(finr_hbm.at[wid], finr_v)
    pltpu.sync_copy(finb_hbm.at[wid], finb_v)
    pltpu.sync_copy(cnt_hbm.at[wid], cnt_v)
    pltpu.sync_copy(mn_hbm, mn_v)
    pltpu.sync_copy(mx_hbm, mx_v)
    c2 = jnp.max(cnt_v[pl.ds(0, _L)])

    @pl.when(c2 > 0)
    def _():
        mnv = mn_v[...]
        mxv = mx_v[...]
        nch = (c2 + _CH - 1) // _CH

        def ch_body(t, c):
            for u in range(_CH // _L):
                rchunk_v[pl.ds(u * _L, _L)] = finr_v[pl.ds(t * _CH + u * _L, _L)]
                bchunk_v[pl.ds(u * _L, _L)] = finb_v[pl.ds(t * _CH + u * _L, _L)]
            pltpu.async_copy(s_hbm.at[bchunk_v], rowbuf_v, sem).wait()

            def row_body(rr, cc):
                for u in range(d // _L):
                    sv = rowbuf_v[rr, pl.ds(u * _L, _L)]
                    rowbuf_v[rr, pl.ds(u * _L, _L)] = mnv + (mxv - mnv) * sv
                return cc

            lax.fori_loop(0, _CH, row_body, 0)
            pltpu.async_copy(rowbuf_v, out_ref.at[rchunk_v], sem).wait()
            return c

        lax.fori_loop(0, nch, ch_body, 0)


def _s2_scatter(out_mutref, finr, finb, cnt, s, mn, mx):
    d = s.shape[1]
    mesh = plsc.VectorSubcoreMesh(core_axis_name="c", subcore_axis_name="s")
    body = functools.partial(_s2_body, d)
    pl.kernel(
        body,
        out_type=(),
        mesh=mesh,
        scratch_types=[
            pltpu.VMEM((_FW,), jnp.int32),
            pltpu.VMEM((_FW,), jnp.int32),
            pltpu.VMEM((_L,), jnp.int32),
            pltpu.VMEM((_L,), jnp.float32),
            pltpu.VMEM((_L,), jnp.float32),
            pltpu.VMEM((_CH,), jnp.int32),
            pltpu.VMEM((_CH,), jnp.int32),
            pltpu.VMEM((_CH, d), jnp.float32),
            pltpu.SemaphoreType.DMA,
        ],
        compiler_params=pltpu.CompilerParams(needs_layout_passes=False),
    )(out_mutref, finr, finb, cnt, s, mn, mx)


def kernel(x, idx, val):
    finr, finb, cnt = _s1_select(idx.astype(jnp.int32), x.shape[0])
    out0, s, mn, mx = _tc_pass(x, val)
    ref = jax.new_ref(out0)
    _s2_scatter(ref, finr, finb, cnt, s, mn, mx)
    return jax.freeze(ref)
